# baseline (speedup 1.0000x reference)
import jax
import jax.numpy as jnp
from jax.experimental import pallas as pl
from jax.experimental.pallas import tpu as pltpu

N = 100000
D_IN = 128
D_OUT = 20
EPS = 1e-5
CH = 9088
NF = 11
TAIL = N - NF * CH


def _col(v_ref):
    return v_ref[...].reshape(1, D_OUT).T


def _body(x_hbm, w_ref, b_ref, g_ref, be_ref, o_ref,
          h_ref, h_tail, xbuf, xtail, stage, stage_t,
          s1, s2, bb, sb, sem0, sem1, osem0, osem1):
    sems = (sem0, sem1)
    osems = (osem0, osem1)

    def x_copy(i):
        return pltpu.make_async_copy(
            x_hbm.at[pl.ds(i * CH, CH), :], xbuf.at[i % 2], sems[i % 2])

    tail_copy = pltpu.make_async_copy(
        x_hbm.at[pl.ds(NF * CH, TAIL), :], xtail, sems[NF % 2])

    s1[...] = jnp.zeros_like(s1)
    s2[...] = jnp.zeros_like(s2)
    bb[...] = jnp.broadcast_to(_col(b_ref), (D_OUT, CH))

    x_copy(0).start()
    for i in range(NF):
        if i + 1 < NF:
            x_copy(i + 1).start()
        elif i + 1 == NF:
            tail_copy.start()
        x_copy(i).wait()
        z = jax.lax.dot_general(
            w_ref[...], xbuf[i % 2],
            (((1,), (1,)), ((), ())),
            preferred_element_type=jnp.float32,
        )
        h = jnp.tanh(z + bb[...])
        h_ref[i] = h
        s1[...] += jnp.sum(h, axis=1, keepdims=True)
        s2[...] += jnp.sum(h * h, axis=1, keepdims=True)

    tail_copy.wait()
    zt = jax.lax.dot_general(
        w_ref[...], xtail[...],
        (((1,), (1,)), ((), ())),
        preferred_element_type=jnp.float32,
    )
    ht = jnp.tanh(zt + bb[:, :TAIL])
    h_tail[...] = ht
    s1[...] += jnp.sum(ht, axis=1, keepdims=True)
    s2[...] += jnp.sum(ht * ht, axis=1, keepdims=True)

    mean = s1[...] * (1.0 / N)
    var = s2[...] * (1.0 / N) - mean * mean
    inv = jax.lax.rsqrt(var + EPS) * _col(g_ref)
    shift = _col(be_ref) - mean * inv
    bb[...] = jnp.broadcast_to(inv, (D_OUT, CH))
    sb[...] = jnp.broadcast_to(shift, (D_OUT, CH))

    out_cps = [None] * NF
    for j in range(NF):
        slot = j % 2
        if j >= 2:
            out_cps[j - 2].wait()
        stage[slot] = h_ref[j] * bb[...] + sb[...]
        cp = pltpu.make_async_copy(
            stage.at[slot],
            o_ref.at[:, pl.ds(j * CH, CH)],
            osems[slot],
        )
        cp.start()
        out_cps[j] = cp
    out_cps[NF - 2].wait()
    out_cps[NF - 1].wait()

    stage_t[...] = h_tail[...] * bb[:, :TAIL] + sb[:, :TAIL]
    tcp = pltpu.make_async_copy(
        stage_t, o_ref.at[:, pl.ds(NF * CH, TAIL)], osems[0])
    tcp.start()
    tcp.wait()


def kernel(x, W, b, gamma, beta):
    yt = pl.pallas_call(
        _body,
        grid=(1,),
        in_specs=[
            pl.BlockSpec(memory_space=pl.ANY),
            pl.BlockSpec((D_OUT, D_IN), lambda i: (0, 0)),
            pl.BlockSpec((D_OUT,), lambda i: (0,)),
            pl.BlockSpec((D_OUT,), lambda i: (0,)),
            pl.BlockSpec((D_OUT,), lambda i: (0,)),
        ],
        out_specs=pl.BlockSpec(memory_space=pl.ANY),
        out_shape=jax.ShapeDtypeStruct((D_OUT, N), jnp.float32),
        scratch_shapes=[
            pltpu.VMEM((NF, D_OUT, CH), jnp.float32),
            pltpu.VMEM((D_OUT, TAIL), jnp.float32),
            pltpu.VMEM((2, CH, D_IN), jnp.float32),
            pltpu.VMEM((TAIL, D_IN), jnp.float32),
            pltpu.VMEM((2, D_OUT, CH), jnp.float32),
            pltpu.VMEM((D_OUT, TAIL), jnp.float32),
            pltpu.VMEM((D_OUT, 1), jnp.float32),
            pltpu.VMEM((D_OUT, 1), jnp.float32),
            pltpu.VMEM((D_OUT, CH), jnp.float32),
            pltpu.VMEM((D_OUT, CH), jnp.float32),
            pltpu.SemaphoreType.DMA,
            pltpu.SemaphoreType.DMA,
            pltpu.SemaphoreType.DMA,
            pltpu.SemaphoreType.DMA,
        ],
    )(x, W, b, gamma, beta)
    return yt.T


# speedup vs baseline: 1.1007x; 1.1007x over previous
import jax
import jax.numpy as jnp
from jax.experimental import pallas as pl
from jax.experimental.pallas import tpu as pltpu

N = 100000
D_IN = 128
D_OUT = 20
EPS = 1e-5
BLK = 10000
NB = N // BLK


def _col(v_ref):
    return v_ref[...].reshape(1, D_OUT).T


def _body(x_hbm, w_ref, b_ref, g_ref, be_ref, o_ref,
          h_ref, xbuf, s1, s2, bb, sb, sem0, sem1, sem2):
    sems = (sem0, sem1, sem2)

    def x_copy(i):
        slot = i % 3
        return pltpu.make_async_copy(
            x_hbm.at[pl.ds(i * BLK, BLK), :], xbuf.at[slot], sems[slot])

    s1[...] = jnp.zeros_like(s1)
    s2[...] = jnp.zeros_like(s2)
    bb[...] = jnp.broadcast_to(_col(b_ref), (D_OUT, BLK))

    x_copy(0).start()
    x_copy(1).start()
    for i in range(NB):
        if i + 2 < NB:
            x_copy(i + 2).start()
        x_copy(i).wait()
        z = jax.lax.dot_general(
            w_ref[...], xbuf[i % 3],
            (((1,), (1,)), ((), ())),
            preferred_element_type=jnp.float32,
        )
        h = jnp.tanh(z + bb[...])
        h_ref[i] = h
        s1[...] += jnp.sum(h, axis=1, keepdims=True)
        s2[...] += jnp.sum(h * h, axis=1, keepdims=True)

    mean = s1[...] * (1.0 / N)
    var = s2[...] * (1.0 / N) - mean * mean
    inv = jax.lax.rsqrt(var + EPS) * _col(g_ref)
    shift = _col(be_ref) - mean * inv
    bb[...] = jnp.broadcast_to(inv, (D_OUT, BLK))
    sb[...] = jnp.broadcast_to(shift, (D_OUT, BLK))
    for j in range(NB):
        o_ref[:, j * BLK:(j + 1) * BLK] = h_ref[j] * bb[...] + sb[...]


def kernel(x, W, b, gamma, beta):
    yt = pl.pallas_call(
        _body,
        grid=(1,),
        in_specs=[
            pl.BlockSpec(memory_space=pl.ANY),
            pl.BlockSpec((D_OUT, D_IN), lambda i: (0, 0)),
            pl.BlockSpec((D_OUT,), lambda i: (0,)),
            pl.BlockSpec((D_OUT,), lambda i: (0,)),
            pl.BlockSpec((D_OUT,), lambda i: (0,)),
        ],
        out_specs=pl.BlockSpec((D_OUT, N), lambda i: (0, 0)),
        out_shape=jax.ShapeDtypeStruct((D_OUT, N), jnp.float32),
        scratch_shapes=[
            pltpu.VMEM((NB, D_OUT, BLK), jnp.float32),
            pltpu.VMEM((3, BLK, D_IN), jnp.float32),
            pltpu.VMEM((D_OUT, 1), jnp.float32),
            pltpu.VMEM((D_OUT, 1), jnp.float32),
            pltpu.VMEM((D_OUT, BLK), jnp.float32),
            pltpu.VMEM((D_OUT, BLK), jnp.float32),
            pltpu.SemaphoreType.DMA,
            pltpu.SemaphoreType.DMA,
            pltpu.SemaphoreType.DMA,
        ],
    )(x, W, b, gamma, beta)
    return yt.T
